# baseline (speedup 1.0000x reference)
import functools

import jax
import jax.numpy as jnp
from jax.experimental import pallas as pl

N = 50000
D = 256
S = 6
BN = 1000


def _project_tables_kernel(emb_ref, w0_ref, w1_ref, t0_ref, t1_ref):
    emb = emb_ref[...]
    scale = 1.0 / S
    for w_ref, t_ref in ((w0_ref, t0_ref), (w1_ref, t1_ref)):
        wb = w_ref[:, D:]
        t = jax.lax.dot_general(
            emb, wb, (((1,), (1,)), ((), ())),
            preferred_element_type=jnp.float32,
            precision=jax.lax.Precision.HIGHEST,
        )
        t_ref[...] = (t * scale).astype(jnp.bfloat16)


def _sage_kernel(x0_ref, nb_ref, w0a_ref, w1a_ref, t0_ref, t1_ref, out_ref):
    nb = nb_ref[...]
    col_ids = jax.lax.broadcasted_iota(jnp.int32, (BN, D), 1).astype(jnp.bfloat16)
    one = jnp.ones((BN, D), jnp.bfloat16)
    zero = jnp.zeros((BN, D), jnp.bfloat16)
    counts = zero
    for s in range(S):
        counts = counts + jnp.where(nb[:, s][:, None] == col_ids, one, zero)

    emb = x0_ref[...]
    for layer, (wa_ref, t_ref) in enumerate(((w0a_ref, t0_ref), (w1a_ref, t1_ref))):
        h = jax.lax.dot_general(
            emb, wa_ref[...], (((1,), (1,)), ((), ())),
            preferred_element_type=jnp.float32,
        )
        h = h + jax.lax.dot_general(
            counts, t_ref[...], (((1,), (0,)), ((), ())),
            preferred_element_type=jnp.float32,
        )
        h = jnp.maximum(h, 0.0)
        if layer == 0:
            emb = h.astype(jnp.bfloat16)
    out_ref[...] = h


@jax.jit
def kernel(x0, emb_table, W0, W1, neighbors):
    nb = neighbors.astype(jnp.bfloat16)
    x0b = x0.astype(jnp.bfloat16)
    w0a = W0[:, :D].astype(jnp.bfloat16)
    w1a = W1[:, :D].astype(jnp.bfloat16)
    t0, t1 = pl.pallas_call(
        _project_tables_kernel,
        out_shape=(
            jax.ShapeDtypeStruct((D, D), jnp.bfloat16),
            jax.ShapeDtypeStruct((D, D), jnp.bfloat16),
        ),
    )(emb_table, W0, W1)

    grid = N // BN
    small = pl.BlockSpec((D, D), lambda i: (0, 0))
    out = pl.pallas_call(
        _sage_kernel,
        grid=(grid,),
        in_specs=[
            pl.BlockSpec((BN, D), lambda i: (i, 0)),
            pl.BlockSpec((BN, S), lambda i: (i, 0)),
            small, small, small, small,
        ],
        out_specs=pl.BlockSpec((BN, D), lambda i: (i, 0)),
        out_shape=jax.ShapeDtypeStruct((N, D), jnp.float32),
    )(x0b, nb, w0a, w1a, t0, t1)
    return out


# speedup vs baseline: 1.2237x; 1.2237x over previous
import functools

import jax
import jax.numpy as jnp
from jax.experimental import pallas as pl

N = 50000
D = 256
S = 6
BN = 1000
SUB = 4
BS = BN // SUB


def _project_tables_kernel(emb_ref, w0_ref, w1_ref, t0_ref, t1_ref):
    emb = emb_ref[...]
    scale = 1.0 / S
    for w_ref, t_ref in ((w0_ref, t0_ref), (w1_ref, t1_ref)):
        wb = w_ref[:, D:]
        t = jax.lax.dot_general(
            emb, wb, (((1,), (1,)), ((), ())),
            preferred_element_type=jnp.float32,
            precision=jax.lax.Precision.HIGHEST,
        )
        t_ref[...] = t * scale


def _sage_kernel(x0_ref, nb_ref, w0_ref, w1_ref, t0_ref, t1_ref, out_ref):
    col_ids = jax.lax.broadcasted_iota(jnp.int32, (BS, D), 1)
    w0a = w0_ref[:, :D]
    w1a = w1_ref[:, :D]
    t0 = t0_ref[...]
    t1 = t1_ref[...]
    for b in range(SUB):
        rows = pl.ds(b * BS, BS)
        nb = nb_ref[rows, :]
        counts = jnp.zeros((BS, D), jnp.float32)
        for s in range(S):
            counts = counts + (nb[:, s][:, None] == col_ids).astype(jnp.float32)

        emb = x0_ref[rows, :]
        for layer, (wa, t) in enumerate(((w0a, t0), (w1a, t1))):
            h = jax.lax.dot_general(
                emb, wa, (((1,), (1,)), ((), ())),
                preferred_element_type=jnp.float32,
            )
            h = h + jax.lax.dot_general(
                counts, t, (((1,), (0,)), ((), ())),
                preferred_element_type=jnp.float32,
            )
            emb = jnp.maximum(h, 0.0)
        out_ref[rows, :] = emb


@jax.jit
def kernel(x0, emb_table, W0, W1, neighbors):
    nb = neighbors.astype(jnp.int32)
    t0, t1 = pl.pallas_call(
        _project_tables_kernel,
        out_shape=(
            jax.ShapeDtypeStruct((D, D), jnp.float32),
            jax.ShapeDtypeStruct((D, D), jnp.float32),
        ),
    )(emb_table, W0, W1)

    grid = N // BN
    full = pl.BlockSpec((D, 2 * D), lambda i: (0, 0))
    small = pl.BlockSpec((D, D), lambda i: (0, 0))
    out = pl.pallas_call(
        _sage_kernel,
        grid=(grid,),
        in_specs=[
            pl.BlockSpec((BN, D), lambda i: (i, 0)),
            pl.BlockSpec((BN, S), lambda i: (i, 0)),
            full, full, small, small,
        ],
        out_specs=pl.BlockSpec((BN, D), lambda i: (i, 0)),
        out_shape=jax.ShapeDtypeStruct((N, D), jnp.float32),
    )(x0, nb, W0, W1, t0, t1)
    return out


# speedup vs baseline: 1.4769x; 1.2069x over previous
import functools

import jax
import jax.numpy as jnp
from jax.experimental import pallas as pl

N = 50000
D = 256
S = 6
BN = 2000
SUB = 10
BS = BN // SUB


def _project_tables_kernel(emb_ref, w0_ref, w1_ref, t0_ref, t1_ref,
                           w0a_ref, w1a_ref):
    emb = emb_ref[...]
    scale = 1.0 / S
    for w_ref, t_ref, wa_ref in ((w0_ref, t0_ref, w0a_ref),
                                 (w1_ref, t1_ref, w1a_ref)):
        wb = w_ref[:, D:]
        t = jax.lax.dot_general(
            emb, wb, (((1,), (1,)), ((), ())),
            preferred_element_type=jnp.float32,
            precision=jax.lax.Precision.HIGHEST,
        )
        t_ref[...] = (t * scale).astype(jnp.bfloat16)
        wa_ref[...] = w_ref[:, :D].astype(jnp.bfloat16)


def _sage_kernel(x0_ref, nb_ref, w0a_ref, w1a_ref, t0_ref, t1_ref, out_ref):
    col_ids = jax.lax.broadcasted_iota(jnp.int32, (BS, D), 1)
    w0a = w0a_ref[...]
    w1a = w1a_ref[...]
    t0 = t0_ref[...]
    t1 = t1_ref[...]
    for b in range(SUB):
        rows = pl.ds(b * BS, BS)
        nb = nb_ref[rows, :]
        counts = jnp.zeros((BS, D), jnp.float32)
        for s in range(S):
            counts = counts + (nb[:, s][:, None] == col_ids).astype(jnp.float32)
        counts = counts.astype(jnp.bfloat16)

        emb = x0_ref[rows, :].astype(jnp.bfloat16)
        for layer, (wa, t) in enumerate(((w0a, t0), (w1a, t1))):
            h = jax.lax.dot_general(
                emb, wa, (((1,), (1,)), ((), ())),
                preferred_element_type=jnp.float32,
            )
            h = h + jax.lax.dot_general(
                counts, t, (((1,), (0,)), ((), ())),
                preferred_element_type=jnp.float32,
            )
            h = jnp.maximum(h, 0.0)
            if layer == 0:
                emb = h.astype(jnp.bfloat16)
        out_ref[rows, :] = h


@jax.jit
def kernel(x0, emb_table, W0, W1, neighbors):
    nb = neighbors.astype(jnp.int32)
    t0, t1, w0a, w1a = pl.pallas_call(
        _project_tables_kernel,
        out_shape=(
            jax.ShapeDtypeStruct((D, D), jnp.bfloat16),
            jax.ShapeDtypeStruct((D, D), jnp.bfloat16),
            jax.ShapeDtypeStruct((D, D), jnp.bfloat16),
            jax.ShapeDtypeStruct((D, D), jnp.bfloat16),
        ),
    )(emb_table, W0, W1)

    grid = N // BN
    small = pl.BlockSpec((D, D), lambda i: (0, 0))
    out = pl.pallas_call(
        _sage_kernel,
        grid=(grid,),
        in_specs=[
            pl.BlockSpec((BN, D), lambda i: (i, 0)),
            pl.BlockSpec((BN, S), lambda i: (i, 0)),
            small, small, small, small,
        ],
        out_specs=pl.BlockSpec((BN, D), lambda i: (i, 0)),
        out_shape=jax.ShapeDtypeStruct((N, D), jnp.float32),
    )(x0, nb, w0a, w1a, t0, t1)
    return out
